# baseline (speedup 1.0000x reference)
import functools

import jax
import jax.numpy as jnp
from jax import lax
from jax.experimental import pallas as pl
from jax.experimental.pallas import tpu as pltpu
from jax.experimental.pallas import tpu_sc as plsc

VOCAB2 = 1000002
EMBED = 64
B_FLAT = 4096 * 200
NC, NS = 2, 16
NW = NC * NS
PER_W = B_FLAT // NW
CHUNK = 128
NCHUNK = PER_W // CHUNK


def _make_gather():
    mesh = plsc.VectorSubcoreMesh(core_axis_name="c", subcore_axis_name="s")

    @functools.partial(
        pl.kernel,
        out_type=jax.ShapeDtypeStruct((B_FLAT, EMBED), jnp.float32),
        mesh=mesh,
        scratch_types=[
            pltpu.VMEM((NCHUNK, CHUNK), jnp.int32),
            pltpu.VMEM((CHUNK, EMBED), jnp.float32),
            pltpu.SemaphoreType.DMA,
        ],
        compiler_params=pltpu.CompilerParams(use_tc_tiling_on_sc=False),
    )
    def gather_kernel(idx_hbm, table_hbm, out_hbm, idx_v, rows_v, sem):
        wid = lax.axis_index("s") * NC + lax.axis_index("c")
        pltpu.sync_copy(idx_hbm.at[wid], idx_v)

        def step(j, carry):
            pltpu.async_copy(table_hbm.at[idx_v.at[j]], rows_v, sem).wait()
            base = wid * PER_W + j * CHUNK
            pltpu.sync_copy(rows_v, out_hbm.at[pl.ds(base, CHUNK)])
            return carry

        lax.fori_loop(0, NCHUNK, step, 0)

    return gather_kernel


_gather = _make_gather()


@jax.jit
def kernel(word_batch, table):
    flat = word_batch.reshape(-1).astype(jnp.int32)
    idx3 = flat.reshape(NW, NCHUNK, CHUNK)
    out = _gather(idx3, table)
    return out.reshape(word_batch.shape[0], word_batch.shape[1], EMBED)


# speedup vs baseline: 1.1139x; 1.1139x over previous
import functools

import jax
import jax.numpy as jnp
from jax import lax
from jax.experimental import pallas as pl
from jax.experimental.pallas import tpu as pltpu
from jax.experimental.pallas import tpu_sc as plsc

VOCAB2 = 1000002
EMBED = 64
B_FLAT = 4096 * 200
NC, NS = 2, 16
NW = NC * NS
PER_W = B_FLAT // NW
CHUNK = 128
NCHUNK = PER_W // CHUNK
NBUF = 8
NGROUP = NCHUNK // NBUF


def _make_gather():
    mesh = plsc.VectorSubcoreMesh(core_axis_name="c", subcore_axis_name="s")

    @functools.partial(
        pl.kernel,
        out_type=jax.ShapeDtypeStruct((B_FLAT, EMBED), jnp.float32),
        mesh=mesh,
        scratch_types=[
            pltpu.VMEM((NCHUNK, CHUNK), jnp.int32),
            pltpu.VMEM((NBUF, CHUNK, EMBED), jnp.float32),
            pltpu.SemaphoreType.DMA((NBUF,)),
            pltpu.SemaphoreType.DMA((NBUF,)),
        ],
        compiler_params=pltpu.CompilerParams(use_tc_tiling_on_sc=False),
    )
    def gather_kernel(idx_hbm, table_hbm, out_hbm, idx_v, rows_v, gsem, ssem):
        wid = lax.axis_index("s") * NC + lax.axis_index("c")
        out_base = wid * PER_W
        pltpu.sync_copy(idx_hbm.at[wid], idx_v)

        for b in range(NBUF):
            pltpu.async_copy(table_hbm.at[idx_v.at[b]], rows_v.at[b],
                             gsem.at[b])

        def group(g, carry):
            j0 = g * NBUF
            for b in range(NBUF):
                j = j0 + b
                pltpu.make_async_copy(table_hbm.at[idx_v.at[j]], rows_v.at[b],
                                      gsem.at[b]).wait()
                pltpu.async_copy(rows_v.at[b],
                                 out_hbm.at[pl.ds(out_base + j * CHUNK, CHUNK)],
                                 ssem.at[b])
                pltpu.make_async_copy(
                    rows_v.at[b],
                    out_hbm.at[pl.ds(out_base + j * CHUNK, CHUNK)],
                    ssem.at[b]).wait()

                @pl.when(g < NGROUP - 1)
                def _():
                    jn = j + NBUF
                    pltpu.async_copy(table_hbm.at[idx_v.at[jn]], rows_v.at[b],
                                     gsem.at[b])

            return carry

        lax.fori_loop(0, NGROUP, group, 0)

    return gather_kernel


_gather = _make_gather()


@jax.jit
def kernel(word_batch, table):
    flat = word_batch.reshape(-1).astype(jnp.int32)
    idx3 = flat.reshape(NW, NCHUNK, CHUNK)
    out = _gather(idx3, table)
    return out.reshape(word_batch.shape[0], word_batch.shape[1], EMBED)


# speedup vs baseline: 1.3631x; 1.2236x over previous
import functools

import jax
import jax.numpy as jnp
from jax import lax
from jax.experimental import pallas as pl
from jax.experimental.pallas import tpu as pltpu
from jax.experimental.pallas import tpu_sc as plsc

VOCAB2 = 1000002
EMBED = 64
WIDE = 128
B_FLAT = 4096 * 200
NC, NS = 2, 16
NW = NC * NS
PER_W = B_FLAT // NW
CHUNK = 128
NCHUNK = PER_W // CHUNK
NBUF = 6
NGROUP = NCHUNK // NBUF
NTAIL = NCHUNK - NGROUP * NBUF


def _make_gather():
    mesh = plsc.VectorSubcoreMesh(core_axis_name="c", subcore_axis_name="s")

    @functools.partial(
        pl.kernel,
        out_type=jax.ShapeDtypeStruct((B_FLAT, WIDE), jnp.float32),
        mesh=mesh,
        scratch_types=[
            pltpu.VMEM((NCHUNK, CHUNK), jnp.int32),
            pltpu.VMEM((NBUF, CHUNK, WIDE), jnp.float32),
            pltpu.SemaphoreType.DMA((NBUF,)),
            pltpu.SemaphoreType.DMA((NBUF,)),
        ],
    )
    def gather_kernel(idx_hbm, table_hbm, out_hbm, idx_v, rows_v, gsem, ssem):
        wid = lax.axis_index("s") * NC + lax.axis_index("c")
        out_base = wid * PER_W
        pltpu.sync_copy(idx_hbm.at[wid], idx_v)

        for b in range(NBUF):
            pltpu.async_copy(table_hbm.at[idx_v.at[b]], rows_v.at[b],
                             gsem.at[b])

        def step(j, b):
            pltpu.make_async_copy(table_hbm.at[idx_v.at[j]], rows_v.at[b],
                                  gsem.at[b]).wait()
            dst = out_hbm.at[pl.ds(out_base + j * CHUNK, CHUNK)]
            src = rows_v.at[b]
            pltpu.async_copy(src, dst, ssem.at[b])
            pltpu.make_async_copy(src, dst, ssem.at[b]).wait()

            @pl.when(j + NBUF < NCHUNK)
            def _():
                pltpu.async_copy(table_hbm.at[idx_v.at[j + NBUF]],
                                 rows_v.at[b], gsem.at[b])

        def group(g, carry):
            for b in range(NBUF):
                step(g * NBUF + b, b)
            return carry

        lax.fori_loop(0, NGROUP, group, 0)
        for b in range(NTAIL):
            step(NGROUP * NBUF + b, b)

    return gather_kernel


_gather = _make_gather()


@jax.jit
def kernel(word_batch, table):
    flat = word_batch.reshape(-1).astype(jnp.int32)
    idx3 = flat.reshape(NW, NCHUNK, CHUNK)
    wide = jnp.pad(table, ((0, 0), (0, WIDE - EMBED)))
    out = _gather(idx3, wide)
    return out[:, :EMBED].reshape(word_batch.shape[0], word_batch.shape[1], EMBED)
